# baseline (speedup 1.0000x reference)
import functools

import jax
import jax.numpy as jnp
from jax import lax
from jax.experimental import pallas as pl
from jax.experimental.pallas import tpu as pltpu
from jax.experimental.pallas import tpu_sc as plsc

_LANES = 16
_NC = 2
_NS = 16
_NW = _NC * _NS
_CHUNK = 128


def kernel(user_id, item_id, users_table, items_table):
    batch = user_id.shape[0]
    latent = users_table.shape[1]
    bpw = batch // _NW
    n_chunk = bpw // _CHUNK

    uid2 = user_id.astype(jnp.int32).reshape(_NW * n_chunk, _CHUNK)
    iid2 = item_id.astype(jnp.int32).reshape(_NW * n_chunk, _CHUNK)

    @functools.partial(
        pl.kernel,
        out_type=jax.ShapeDtypeStruct((batch,), jnp.float32),
        mesh=plsc.VectorSubcoreMesh(core_axis_name="c", subcore_axis_name="s"),
        compiler_params=pltpu.CompilerParams(
            needs_layout_passes=False, use_tc_tiling_on_sc=False),
        scratch_types=[
            pltpu.VMEM((n_chunk, _CHUNK), jnp.int32),
            pltpu.VMEM((n_chunk, _CHUNK), jnp.int32),
            pltpu.VMEM((bpw, latent), jnp.float32),
            pltpu.VMEM((bpw, latent), jnp.float32),
            pltpu.VMEM((bpw,), jnp.float32),
            pltpu.SemaphoreType.DMA,
        ],
    )
    def mf(uid_hbm, iid_hbm, ut_hbm, it_hbm, out_hbm,
           uidx, iidx, urows, vrows, outv, sem):
        wid = lax.axis_index("s") * _NC + lax.axis_index("c")
        pltpu.sync_copy(uid_hbm.at[pl.ds(wid * n_chunk, n_chunk)], uidx)
        pltpu.sync_copy(iid_hbm.at[pl.ds(wid * n_chunk, n_chunk)], iidx)

        copies = []
        for j in range(n_chunk):
            copies.append(pltpu.async_copy(
                ut_hbm.at[uidx.at[j]],
                urows.at[pl.ds(j * _CHUNK, _CHUNK)], sem))
            copies.append(pltpu.async_copy(
                it_hbm.at[iidx.at[j]],
                vrows.at[pl.ds(j * _CHUNK, _CHUNK)], sem))
        for c in copies:
            c.wait()

        lane = lax.iota(jnp.int32, _LANES)
        cols = [jnp.full((_LANES,), d, jnp.int32) for d in range(latent)]

        def body(g, carry):
            rows = lane + g * _LANES
            acc = jnp.zeros((_LANES,), jnp.float32)
            for d in range(latent):
                u = plsc.load_gather(urows, [rows, cols[d]])
                v = plsc.load_gather(vrows, [rows, cols[d]])
                acc = acc + u * v
            outv[pl.ds(g * _LANES, _LANES)] = acc
            return carry

        lax.fori_loop(0, bpw // _LANES, body, 0)
        pltpu.sync_copy(outv, out_hbm.at[pl.ds(wid * bpw, bpw)])

    return mf(uid2, iid2, users_table, items_table)
